# Initial kernel scaffold; baseline (speedup 1.0000x reference)
import jax
import jax.numpy as jnp
from jax.experimental import pallas as pl


def kernel(x, table):
    raise NotImplementedError("write your pallas kernel here")


# speedup vs baseline: 1.1299x; 1.1299x over previous
import jax
import jax.numpy as jnp
from jax.experimental import pallas as pl

_NUM_PATCHES = 1024
_DIM = 768


def _bcast_body(table_ref, out_ref):
    out_ref[...] = table_ref[...][None]


def kernel(x, table):
    batch = x.shape[0]
    return pl.pallas_call(
        _bcast_body,
        grid=(batch,),
        in_specs=[pl.BlockSpec((_NUM_PATCHES, _DIM), lambda b: (0, 0))],
        out_specs=pl.BlockSpec((1, _NUM_PATCHES, _DIM), lambda b: (b, 0, 0)),
        out_shape=jax.ShapeDtypeStruct((batch, _NUM_PATCHES, _DIM), jnp.float32),
    )(table)
